# baseline (speedup 1.0000x reference)
import functools

import jax
import jax.numpy as jnp
from jax import lax
from jax.experimental import pallas as pl
from jax.experimental.pallas import tpu as pltpu
from jax.experimental.pallas import tpu_sc as plsc

N = 400000
D_IN = 256
D_OUT = 384
B = 8
NC = 2
NS = 16
NW = NC * NS
C = 120
R_SC = 220800
NCHUNK = R_SC // C
LANES = 16
NV = D_IN // LANES
TCC = 3200
G_TC = (N - R_SC) // TCC


def _sc_segment_sums(features, starts16):
    mesh = plsc.VectorSubcoreMesh(core_axis_name="c", subcore_axis_name="s")

    @functools.partial(
        pl.kernel,
        mesh=mesh,
        out_type=jax.ShapeDtypeStruct((NW, B, D_IN), jnp.float32),
        scratch_types=[
            pltpu.VMEM((C, D_IN), jnp.float32),
            pltpu.VMEM((C, D_IN), jnp.float32),
            pltpu.VMEM((C, D_IN), jnp.float32),
            pltpu.VMEM((C, D_IN), jnp.float32),
            pltpu.VMEM((LANES,), jnp.int32),
            pltpu.VMEM((B, D_IN), jnp.float32),
            pltpu.SemaphoreType.DMA,
            pltpu.SemaphoreType.DMA,
            pltpu.SemaphoreType.DMA,
            pltpu.SemaphoreType.DMA,
        ],
    )
    def k(feat_hbm, starts_hbm, out_hbm, fbuf0, fbuf1, fbuf2, fbuf3,
          vbuf, acc, sem0, sem1, sem2, sem3):
        wid = lax.axis_index("c") * NS + lax.axis_index("s")
        nk = (NCHUNK - 1 - wid) // NW + 1

        pltpu.sync_copy(starts_hbm, vbuf)
        svec = vbuf[...]
        bounds = [svec[s] for s in range(B + 1)]

        zero = jnp.zeros((LANES,), jnp.float32)
        for s in range(B):
            for j in range(NV):
                acc[s, pl.ds(j * LANES, LANES)] = zero

        def copy_op(c, buf, sem):
            cr0 = (wid + c * NW) * C
            return pltpu.make_async_copy(
                feat_hbm.at[pl.ds(cr0, C)], buf, sem
            )

        def compute_chunk(c, buf):
            cr0 = (wid + c * NW) * C
            for s in range(B):
                lo = jnp.clip(bounds[s] - cr0, 0, C)
                hi = jnp.clip(bounds[s + 1] - cr0, 0, C)
                @plsc.parallel_loop(lo, hi, unroll=4, carry=(zero,) * NV)
                def part(i, rc, buf=buf):
                    return tuple(
                        rc[j] + buf[i, pl.ds(j * LANES, LANES)]
                        for j in range(NV)
                    )
                for j in range(NV):
                    sl = pl.ds(j * LANES, LANES)
                    acc[s, sl] = acc[s, sl] + part[j]

        bufs = (fbuf0, fbuf1, fbuf2, fbuf3)
        sems = (sem0, sem1, sem2, sem3)
        NBUF = 4

        for c in range(NBUF):
            @pl.when(c < nk)
            def _(c=c):
                copy_op(c, bufs[c], sems[c]).start()

        def ring_body(p, carry):
            for par in range(NBUF):
                c = NBUF * p + par

                @pl.when(c < nk)
                def _(c=c, buf=bufs[par], sem=sems[par]):
                    copy_op(c, buf, sem).wait()
                    compute_chunk(c, buf)

                    @pl.when(c + NBUF < nk)
                    def _():
                        copy_op(c + NBUF, buf, sem).start()
            return carry

        lax.fori_loop(0, (nk + NBUF - 1) // NBUF, ring_body, 0)
        pltpu.sync_copy(acc, out_hbm.at[wid])

    return k(features, starts16)


def _tc_partial_segsum(features, lo8, hi8):
    def k(lo_ref, hi_ref, f_ref, o_ref):
        g = pl.program_id(0)
        r0 = R_SC + g * TCC
        rows = r0 + lax.broadcasted_iota(jnp.int32, (B, TCC), 1)
        onehot_t = jnp.where(
            (rows >= lo_ref[...]) & (rows < hi_ref[...]), 1.0, 0.0
        ).astype(jnp.float32)
        partial = lax.dot_general(
            onehot_t,
            f_ref[...],
            (((1,), (0,)), ((), ())),
            preferred_element_type=jnp.float32,
        )

        @pl.when(g == 0)
        def _():
            o_ref[...] = jnp.zeros_like(o_ref)

        o_ref[...] += partial

    return pl.pallas_call(
        k,
        grid=(G_TC,),
        in_specs=[
            pl.BlockSpec((B, 1), lambda g: (0, 0)),
            pl.BlockSpec((B, 1), lambda g: (0, 0)),
            pl.BlockSpec((TCC, D_IN), lambda g: (R_SC // TCC + g, 0)),
        ],
        out_specs=pl.BlockSpec((B, D_IN), lambda g: (0, 0)),
        out_shape=jax.ShapeDtypeStruct((B, D_IN), jnp.float32),
    )(lo8, hi8, features)


def _tc_combine(partials, tc_partial, W, b_row, counts_col):
    def k(p_ref, t_ref, w_ref, b_ref, c_ref, o_ref):
        seg = jnp.sum(p_ref[...], axis=0) + t_ref[...]
        cnt = c_ref[...]
        proj = jnp.dot(seg, w_ref[...], preferred_element_type=jnp.float32)
        o_ref[...] = (proj + cnt * b_ref[...]) / jnp.maximum(cnt, 1.0)

    return pl.pallas_call(
        k,
        out_shape=jax.ShapeDtypeStruct((B, D_OUT), jnp.float32),
    )(partials, tc_partial, W, b_row, counts_col)


def kernel(features, batch_idx, W, b, batch_size):
    del batch_size
    ids = batch_idx.astype(jnp.int32)
    counts_i = jnp.sum(
        (ids[:, None] == jnp.arange(B, dtype=jnp.int32)[None, :])
        .astype(jnp.int32),
        axis=0,
    )
    cum = jnp.cumsum(counts_i)
    starts16 = jnp.zeros((LANES,), jnp.int32).at[1 : B + 1].set(cum)
    lo8 = starts16[:B].reshape(B, 1)
    hi8 = starts16[1 : B + 1].reshape(B, 1)
    partials = _sc_segment_sums(features, starts16)
    tc_partial = _tc_partial_segsum(features, lo8, hi8)
    counts = counts_i.astype(jnp.float32)
    return _tc_combine(
        partials, tc_partial, W, b.reshape(1, D_OUT), counts.reshape(B, 1)
    )


# speedup vs baseline: 1.0230x; 1.0230x over previous
import functools

import jax
import jax.numpy as jnp
from jax import lax
from jax.experimental import pallas as pl
from jax.experimental.pallas import tpu as pltpu
from jax.experimental.pallas import tpu_sc as plsc

N = 400000
D_IN = 256
D_OUT = 384
B = 8
NC = 2
NS = 16
NW = NC * NS
C = 160
R_SC = 220000
NCHUNK = R_SC // C
LANES = 16
NV = D_IN // LANES
TCC = 4000
G_TC = (N - R_SC) // TCC


def _sc_segment_sums(features, starts16):
    mesh = plsc.VectorSubcoreMesh(core_axis_name="c", subcore_axis_name="s")

    @functools.partial(
        pl.kernel,
        mesh=mesh,
        out_type=jax.ShapeDtypeStruct((NW, B, D_IN), jnp.float32),
        scratch_types=[
            pltpu.VMEM((C, D_IN), jnp.float32),
            pltpu.VMEM((C, D_IN), jnp.float32),
            pltpu.VMEM((C, D_IN), jnp.float32),
            pltpu.VMEM((LANES,), jnp.int32),
            pltpu.VMEM((B, D_IN), jnp.float32),
            pltpu.SemaphoreType.DMA,
            pltpu.SemaphoreType.DMA,
            pltpu.SemaphoreType.DMA,
        ],
    )
    def k(feat_hbm, starts_hbm, out_hbm, fbuf0, fbuf1, fbuf2, vbuf, acc,
          sem0, sem1, sem2):
        wid = lax.axis_index("c") * NS + lax.axis_index("s")
        nk = (NCHUNK - 1 - wid) // NW + 1

        pltpu.sync_copy(starts_hbm, vbuf)
        svec = vbuf[...]
        bounds = [svec[s] for s in range(B + 1)]

        zero = jnp.zeros((LANES,), jnp.float32)
        for s in range(B):
            for j in range(NV):
                acc[s, pl.ds(j * LANES, LANES)] = zero

        def copy_op(c, buf, sem):
            cr0 = (wid + c * NW) * C
            return pltpu.make_async_copy(
                feat_hbm.at[pl.ds(cr0, C)], buf, sem
            )

        def compute_chunk(c, buf):
            cr0 = (wid + c * NW) * C
            for s in range(B):
                lo = jnp.clip(bounds[s] - cr0, 0, C)
                hi = jnp.clip(bounds[s + 1] - cr0, 0, C)
                @plsc.parallel_loop(lo, hi, unroll=4, carry=(zero,) * NV)
                def part(i, rc, buf=buf):
                    return tuple(
                        rc[j] + buf[i, pl.ds(j * LANES, LANES)]
                        for j in range(NV)
                    )
                for j in range(NV):
                    sl = pl.ds(j * LANES, LANES)
                    acc[s, sl] = acc[s, sl] + part[j]

        bufs = (fbuf0, fbuf1, fbuf2)
        sems = (sem0, sem1, sem2)
        NBUF = 3

        for c in range(NBUF):
            @pl.when(c < nk)
            def _(c=c):
                copy_op(c, bufs[c], sems[c]).start()

        def ring_body(p, carry):
            for par in range(NBUF):
                c = NBUF * p + par

                @pl.when(c < nk)
                def _(c=c, buf=bufs[par], sem=sems[par]):
                    copy_op(c, buf, sem).wait()
                    compute_chunk(c, buf)

                    @pl.when(c + NBUF < nk)
                    def _():
                        copy_op(c + NBUF, buf, sem).start()
            return carry

        lax.fori_loop(0, (nk + NBUF - 1) // NBUF, ring_body, 0)
        pltpu.sync_copy(acc, out_hbm.at[wid])

    return k(features, starts16)


def _tc_partial_segsum(features, lo8, hi8):
    def k(lo_ref, hi_ref, f_ref, o_ref):
        g = pl.program_id(0)
        r0 = R_SC + g * TCC
        rows = r0 + lax.broadcasted_iota(jnp.int32, (B, TCC), 1)
        onehot_t = jnp.where(
            (rows >= lo_ref[...]) & (rows < hi_ref[...]), 1.0, 0.0
        ).astype(jnp.float32)
        partial = lax.dot_general(
            onehot_t,
            f_ref[...],
            (((1,), (0,)), ((), ())),
            preferred_element_type=jnp.float32,
        )

        @pl.when(g == 0)
        def _():
            o_ref[...] = jnp.zeros_like(o_ref)

        o_ref[...] += partial

    return pl.pallas_call(
        k,
        grid=(G_TC,),
        in_specs=[
            pl.BlockSpec((B, 1), lambda g: (0, 0)),
            pl.BlockSpec((B, 1), lambda g: (0, 0)),
            pl.BlockSpec((TCC, D_IN), lambda g: (R_SC // TCC + g, 0)),
        ],
        out_specs=pl.BlockSpec((B, D_IN), lambda g: (0, 0)),
        out_shape=jax.ShapeDtypeStruct((B, D_IN), jnp.float32),
    )(lo8, hi8, features)


def _tc_combine(partials, tc_partial, W, b_row, counts_col):
    def k(p_ref, t_ref, w_ref, b_ref, c_ref, o_ref):
        seg = jnp.sum(p_ref[...], axis=0) + t_ref[...]
        cnt = c_ref[...]
        proj = jnp.dot(seg, w_ref[...], preferred_element_type=jnp.float32)
        o_ref[...] = (proj + cnt * b_ref[...]) / jnp.maximum(cnt, 1.0)

    return pl.pallas_call(
        k,
        out_shape=jax.ShapeDtypeStruct((B, D_OUT), jnp.float32),
    )(partials, tc_partial, W, b_row, counts_col)


def kernel(features, batch_idx, W, b, batch_size):
    del batch_size
    ids = batch_idx.astype(jnp.int32)
    counts_i = jnp.sum(
        (ids[:, None] == jnp.arange(B, dtype=jnp.int32)[None, :])
        .astype(jnp.int32),
        axis=0,
    )
    cum = jnp.cumsum(counts_i)
    starts16 = jnp.zeros((LANES,), jnp.int32).at[1 : B + 1].set(cum)
    lo8 = starts16[:B].reshape(B, 1)
    hi8 = starts16[1 : B + 1].reshape(B, 1)
    partials = _sc_segment_sums(features, starts16)
    tc_partial = _tc_partial_segsum(features, lo8, hi8)
    counts = counts_i.astype(jnp.float32)
    return _tc_combine(
        partials, tc_partial, W, b.reshape(1, D_OUT), counts.reshape(B, 1)
    )
